# baseline (speedup 1.0000x reference)
import functools

import jax
import jax.numpy as jnp
from jax import lax
from jax.experimental import pallas as pl
from jax.experimental.pallas import tpu as pltpu
from jax.experimental.pallas import tpu_sc as plsc

BATCH = 16384
HIDDEN = 64
FP_DIM = 167

NC, NS = 2, 16
NW = NC * NS
B_PER_W = BATCH // NW
CHUNK = 128
N_CHUNKS = B_PER_W // CHUNK

BB = 2048


def _gather_body(cid_hbm, eid_hbm, mfc_hbm, mfe_hbm, mlpc_hbm, mlpe_hbm,
                 out_mfc, out_mfe, out_mlpc, out_mlpe,
                 idx_c, idx_e, rows_a, rows_b, sem_a, sem_b):
    wid = lax.axis_index("s") * NC + lax.axis_index("c")
    base = wid * B_PER_W
    for chunk in range(N_CHUNKS):
        off = base + chunk * CHUNK
        pltpu.sync_copy(cid_hbm.at[pl.ds(off, CHUNK)], idx_c)
        pltpu.sync_copy(eid_hbm.at[pl.ds(off, CHUNK)], idx_e)
        cp_a = pltpu.async_copy(mfc_hbm.at[idx_c], rows_a, sem_a)
        cp_b = pltpu.async_copy(mfe_hbm.at[idx_e], rows_b, sem_b)
        cp_a.wait()
        pltpu.sync_copy(rows_a, out_mfc.at[pl.ds(off, CHUNK)])
        cp_b.wait()
        pltpu.sync_copy(rows_b, out_mfe.at[pl.ds(off, CHUNK)])
        cp_a = pltpu.async_copy(mlpc_hbm.at[idx_c], rows_a, sem_a)
        cp_b = pltpu.async_copy(mlpe_hbm.at[idx_e], rows_b, sem_b)
        cp_a.wait()
        pltpu.sync_copy(rows_a, out_mlpc.at[pl.ds(off, CHUNK)])
        cp_b.wait()
        pltpu.sync_copy(rows_b, out_mlpe.at[pl.ds(off, CHUNK)])


def _sc_gather(compound_ids, enzyme_ids, mf_c16, mf_e16, mlp_c16, mlp_e16):
    mesh = plsc.VectorSubcoreMesh(core_axis_name="c", subcore_axis_name="s")
    row = jax.ShapeDtypeStruct((BATCH, HIDDEN), jnp.bfloat16)
    fn = pl.kernel(
        _gather_body,
        out_type=(row, row, row, row),
        mesh=mesh,
        compiler_params=pltpu.CompilerParams(use_tc_tiling_on_sc=False),
        scratch_types=[
            pltpu.VMEM((CHUNK,), jnp.int32),
            pltpu.VMEM((CHUNK,), jnp.int32),
            pltpu.VMEM((CHUNK, HIDDEN), jnp.bfloat16),
            pltpu.VMEM((CHUNK, HIDDEN), jnp.bfloat16),
            pltpu.SemaphoreType.DMA,
            pltpu.SemaphoreType.DMA,
        ],
    )
    return fn(compound_ids, enzyme_ids, mf_c16, mf_e16, mlp_c16, mlp_e16)


def _aug_body(augf_ref, w1_ref, b1_ref, w2_ref, b2_ref, out_ref):
    h = jnp.maximum(
        jnp.dot(augf_ref[...], w1_ref[...],
                preferred_element_type=jnp.float32) + b1_ref[...], 0.0)
    out_ref[...] = (
        jnp.dot(h, w2_ref[...], preferred_element_type=jnp.float32)
        + b2_ref[...])


def _aug_mlp(aug_f, aug_W1, aug_b1, aug_W2, aug_b2):
    b1 = aug_b1.reshape(1, HIDDEN)
    b2 = aug_b2.reshape(1, HIDDEN)
    grid = (BATCH // BB,)
    full = lambda shape: pl.BlockSpec(shape, lambda i: (0, 0))
    return pl.pallas_call(
        _aug_body,
        grid=grid,
        in_specs=[
            pl.BlockSpec((BB, FP_DIM), lambda i: (i, 0)),
            full((FP_DIM, HIDDEN)), full((1, HIDDEN)),
            full((HIDDEN, HIDDEN)), full((1, HIDDEN)),
        ],
        out_specs=pl.BlockSpec((BB, HIDDEN), lambda i: (i, 0)),
        out_shape=jax.ShapeDtypeStruct((BATCH, HIDDEN), jnp.float32),
    )(aug_f, aug_W1, b1, aug_W2, b2)


def _fuse_body(mfc_ref, mfe_ref, mlpc_ref, mlpe_ref, aug_ref,
               fA_ref, fB_ref, fb_ref, wmf_ref, wmlp_ref, waug_ref, cb_ref,
               out_ref):
    mfc = mfc_ref[...].astype(jnp.float32)
    mfe = mfe_ref[...].astype(jnp.float32)
    mf = mfe * mfc
    mlp = jnp.maximum(
        jnp.dot(mlpe_ref[...].astype(jnp.float32), fA_ref[...],
                preferred_element_type=jnp.float32)
        + jnp.dot(mlpc_ref[...].astype(jnp.float32), fB_ref[...],
                  preferred_element_type=jnp.float32)
        + fb_ref[...], 0.0)
    logits = (jnp.dot(mf, wmf_ref[...], preferred_element_type=jnp.float32)
              + jnp.dot(mlp, wmlp_ref[...], preferred_element_type=jnp.float32)
              + jnp.dot(aug_ref[...], waug_ref[...],
                        preferred_element_type=jnp.float32)
              + cb_ref[0, 0])
    out_ref[...] = jax.nn.sigmoid(logits)


def _tc_fuse(mfc_rows, mfe_rows, mlpc_rows, mlpe_rows, aug,
             fc1_W, fc1_b, ce_W, ce_b):
    fA = fc1_W[:HIDDEN, :]
    fB = fc1_W[HIDDEN:, :]
    wmf = ce_W[0:HIDDEN, :]
    wmlp = ce_W[HIDDEN:2 * HIDDEN, :]
    waug = ce_W[2 * HIDDEN:, :]
    fb = fc1_b.reshape(1, HIDDEN)
    cb = ce_b.reshape(1, 1)

    grid = (BATCH // BB,)
    batch_spec = lambda cols: pl.BlockSpec((BB, cols), lambda i: (i, 0))
    full = lambda shape: pl.BlockSpec(shape, lambda i: (0, 0))
    return pl.pallas_call(
        _fuse_body,
        grid=grid,
        in_specs=[
            batch_spec(HIDDEN), batch_spec(HIDDEN),
            batch_spec(HIDDEN), batch_spec(HIDDEN),
            batch_spec(HIDDEN),
            full((HIDDEN, HIDDEN)), full((HIDDEN, HIDDEN)), full((1, HIDDEN)),
            full((HIDDEN, 1)), full((HIDDEN, 1)), full((HIDDEN, 1)),
            full((1, 1)),
        ],
        out_specs=pl.BlockSpec((BB, 1), lambda i: (i, 0)),
        out_shape=jax.ShapeDtypeStruct((BATCH, 1), jnp.float32),
    )(mfc_rows, mfe_rows, mlpc_rows, mlpe_rows, aug,
      fA, fB, fb, wmf, wmlp, waug, cb)


def kernel(compound_ids, enzyme_ids, aug_f, aug_W1, aug_b1, aug_W2, aug_b2,
           mf_c_table, mf_e_table, mlp_c_table, mlp_e_table,
           fc1_W, fc1_b, ce_W, ce_b):
    mf_c16 = mf_c_table.astype(jnp.bfloat16)
    mf_e16 = mf_e_table.astype(jnp.bfloat16)
    mlp_c16 = mlp_c_table.astype(jnp.bfloat16)
    mlp_e16 = mlp_e_table.astype(jnp.bfloat16)
    mfc_rows, mfe_rows, mlpc_rows, mlpe_rows = _sc_gather(
        compound_ids, enzyme_ids, mf_c16, mf_e16, mlp_c16, mlp_e16)
    aug = _aug_mlp(aug_f, aug_W1, aug_b1, aug_W2, aug_b2)
    return _tc_fuse(mfc_rows, mfe_rows, mlpc_rows, mlpe_rows, aug,
                    fc1_W, fc1_b, ce_W, ce_b)


# speedup vs baseline: 2.5908x; 2.5908x over previous
import math

import jax
import jax.numpy as jnp
from jax import lax
from jax.experimental import pallas as pl
from jax.experimental.pallas import tpu as pltpu
from jax.experimental.pallas import tpu_sc as plsc

BATCH = 16384
HIDDEN = 64
FP_DIM = 167

NC, NS = 2, 16
NW = NC * NS
B_PER_W = BATCH // NW
CHUNK = 128
N_CHUNKS = B_PER_W // CHUNK

CB = 4096
BB = 2048


def _prep_body(a_ref, b_ref, out_ref):
    at = jnp.transpose(a_ref[...], (1, 0))
    bt = jnp.transpose(b_ref[...], (1, 0))
    out_ref[...] = jnp.concatenate([at, bt], axis=1)


def _prep_pair(ta, tb, n_rows):
    grid = (math.ceil(n_rows / CB),)
    return pl.pallas_call(
        _prep_body,
        grid=grid,
        in_specs=[
            pl.BlockSpec((HIDDEN, CB), lambda i: (0, i)),
            pl.BlockSpec((HIDDEN, CB), lambda i: (0, i)),
        ],
        out_specs=pl.BlockSpec((CB, 2 * HIDDEN), lambda i: (i, 0)),
        out_shape=jax.ShapeDtypeStruct((n_rows, 2 * HIDDEN), jnp.float32),
    )(ta.T, tb.T)


def _gather_body(cid_hbm, eid_hbm, comb_c_hbm, comb_e_hbm,
                 out_c, out_e, idx_c, idx_e, rows_a, rows_b, sem_a, sem_b):
    wid = lax.axis_index("s") * NC + lax.axis_index("c")
    base = wid * B_PER_W
    for chunk in range(N_CHUNKS):
        off = base + chunk * CHUNK
        pltpu.sync_copy(cid_hbm.at[pl.ds(off, CHUNK)], idx_c)
        pltpu.sync_copy(eid_hbm.at[pl.ds(off, CHUNK)], idx_e)
        cp_a = pltpu.async_copy(comb_c_hbm.at[idx_c], rows_a, sem_a)
        cp_b = pltpu.async_copy(comb_e_hbm.at[idx_e], rows_b, sem_b)
        cp_a.wait()
        pltpu.sync_copy(rows_a, out_c.at[pl.ds(off, CHUNK)])
        cp_b.wait()
        pltpu.sync_copy(rows_b, out_e.at[pl.ds(off, CHUNK)])


def _sc_gather(compound_ids, enzyme_ids, comb_c, comb_e):
    mesh = plsc.VectorSubcoreMesh(core_axis_name="c", subcore_axis_name="s")
    out = jax.ShapeDtypeStruct((BATCH, 2 * HIDDEN), jnp.float32)
    fn = pl.kernel(
        _gather_body,
        out_type=(out, out),
        mesh=mesh,
        scratch_types=[
            pltpu.VMEM((CHUNK,), jnp.int32),
            pltpu.VMEM((CHUNK,), jnp.int32),
            pltpu.VMEM((CHUNK, 2 * HIDDEN), jnp.float32),
            pltpu.VMEM((CHUNK, 2 * HIDDEN), jnp.float32),
            pltpu.SemaphoreType.DMA,
            pltpu.SemaphoreType.DMA,
        ],
    )
    return fn(compound_ids, enzyme_ids, comb_c, comb_e)


def _aug_body(augf_ref, w1_ref, b1_ref, w2_ref, b2_ref, out_ref):
    h = jnp.maximum(
        jnp.dot(augf_ref[...], w1_ref[...],
                preferred_element_type=jnp.float32) + b1_ref[...], 0.0)
    out_ref[...] = (
        jnp.dot(h, w2_ref[...], preferred_element_type=jnp.float32)
        + b2_ref[...])


def _aug_mlp(aug_f, aug_W1, aug_b1, aug_W2, aug_b2):
    b1 = aug_b1.reshape(1, HIDDEN)
    b2 = aug_b2.reshape(1, HIDDEN)
    grid = (BATCH // BB,)
    full = lambda shape: pl.BlockSpec(shape, lambda i: (0, 0))
    return pl.pallas_call(
        _aug_body,
        grid=grid,
        in_specs=[
            pl.BlockSpec((BB, FP_DIM), lambda i: (i, 0)),
            full((FP_DIM, HIDDEN)), full((1, HIDDEN)),
            full((HIDDEN, HIDDEN)), full((1, HIDDEN)),
        ],
        out_specs=pl.BlockSpec((BB, HIDDEN), lambda i: (i, 0)),
        out_shape=jax.ShapeDtypeStruct((BATCH, HIDDEN), jnp.float32),
    )(aug_f, aug_W1, b1, aug_W2, b2)


def _fuse_body(rc_ref, re_ref, aug_ref,
               fA_ref, fB_ref, fb_ref, wmf_ref, wmlp_ref, waug_ref, cb_ref,
               out_ref):
    mfc = rc_ref[:, :HIDDEN]
    mlpc = rc_ref[:, HIDDEN:]
    mfe = re_ref[:, :HIDDEN]
    mlpe = re_ref[:, HIDDEN:]
    mf = mfe * mfc
    mlp = jnp.maximum(
        jnp.dot(mlpe, fA_ref[...], preferred_element_type=jnp.float32)
        + jnp.dot(mlpc, fB_ref[...], preferred_element_type=jnp.float32)
        + fb_ref[...], 0.0)
    logits = (jnp.dot(mf, wmf_ref[...], preferred_element_type=jnp.float32)
              + jnp.dot(mlp, wmlp_ref[...], preferred_element_type=jnp.float32)
              + jnp.dot(aug_ref[...], waug_ref[...],
                        preferred_element_type=jnp.float32)
              + cb_ref[0, 0])
    out_ref[...] = jax.nn.sigmoid(logits)


def _tc_fuse(rows_c, rows_e, aug, fc1_W, fc1_b, ce_W, ce_b):
    fA = fc1_W[:HIDDEN, :]
    fB = fc1_W[HIDDEN:, :]
    wmf = ce_W[0:HIDDEN, :]
    wmlp = ce_W[HIDDEN:2 * HIDDEN, :]
    waug = ce_W[2 * HIDDEN:, :]
    fb = fc1_b.reshape(1, HIDDEN)
    cb = ce_b.reshape(1, 1)

    grid = (BATCH // BB,)
    full = lambda shape: pl.BlockSpec(shape, lambda i: (0, 0))
    return pl.pallas_call(
        _fuse_body,
        grid=grid,
        in_specs=[
            pl.BlockSpec((BB, 2 * HIDDEN), lambda i: (i, 0)),
            pl.BlockSpec((BB, 2 * HIDDEN), lambda i: (i, 0)),
            pl.BlockSpec((BB, HIDDEN), lambda i: (i, 0)),
            full((HIDDEN, HIDDEN)), full((HIDDEN, HIDDEN)), full((1, HIDDEN)),
            full((HIDDEN, 1)), full((HIDDEN, 1)), full((HIDDEN, 1)),
            full((1, 1)),
        ],
        out_specs=pl.BlockSpec((BB, 1), lambda i: (i, 0)),
        out_shape=jax.ShapeDtypeStruct((BATCH, 1), jnp.float32),
    )(rows_c, rows_e, aug, fA, fB, fb, wmf, wmlp, waug, cb)


def kernel(compound_ids, enzyme_ids, aug_f, aug_W1, aug_b1, aug_W2, aug_b2,
           mf_c_table, mf_e_table, mlp_c_table, mlp_e_table,
           fc1_W, fc1_b, ce_W, ce_b):
    comb_c = _prep_pair(mf_c_table, mlp_c_table, 1000000)
    comb_e = _prep_pair(mf_e_table, mlp_e_table, 100000)
    rows_c, rows_e = _sc_gather(compound_ids, enzyme_ids, comb_c, comb_e)
    aug = _aug_mlp(aug_f, aug_W1, aug_b1, aug_W2, aug_b2)
    return _tc_fuse(rows_c, rows_e, aug, fc1_W, fc1_b, ce_W, ce_b)
